# baseline (speedup 1.0000x reference)
import functools

import jax
import jax.numpy as jnp
from jax import lax
from jax.experimental import pallas as pl
from jax.experimental.pallas import tpu as pltpu
from jax.experimental.pallas import tpu_sc as plsc

_ROWS, _COLS = 16384, 1024
_N = _ROWS * _COLS
_R = _N // 4

_NC, _NS = 2, 16
_NW = _NC * _NS
_B = 128
_K = 16
_RPW = _R // (_B * _NW)
_G = _RPW // _K


_COPY_BLOCK = 1024


def _copy_body(src_ref, dst_ref):
    dst_ref[...] = src_ref[...]


def _tc_copy(x):
    grid = _ROWS // _COPY_BLOCK
    return pl.pallas_call(
        _copy_body,
        grid=(grid,),
        in_specs=[pl.BlockSpec((_COPY_BLOCK, _COLS), lambda i: (i, 0))],
        out_specs=pl.BlockSpec((_COPY_BLOCK, _COLS), lambda i: (i, 0)),
        out_shape=jax.ShapeDtypeStruct((_ROWS, _COLS), jnp.float32),
    )(x)


_CH = _K * _B


def _sc_scatter_body(out_ref, idx_ref, par_ref, idx_v, par_v, sem):
    c = lax.axis_index("c")
    s = lax.axis_index("s")
    wid = s * _NC + c
    base = wid * (_R // _NW)

    def group(g, carry):
        e = base + g * _CH
        pltpu.sync_copy(idx_ref.at[pl.ds(e, _CH)], idx_v)
        pltpu.sync_copy(par_ref.at[pl.ds(e, _CH)], par_v)
        pltpu.async_copy(par_v, out_ref.at[idx_v], sem).wait()
        return carry

    lax.fori_loop(0, _G, group, None)


def _make_sc_scatter():
    mesh = plsc.VectorSubcoreMesh(
        core_axis_name="c", subcore_axis_name="s",
        num_cores=_NC, num_subcores=_NS,
    )
    return pl.kernel(
        _sc_scatter_body,
        out_type=(),
        mesh=mesh,
        scratch_types=[
            pltpu.VMEM((_CH,), jnp.int32),
            pltpu.VMEM((_CH,), jnp.float32),
            pltpu.SemaphoreType.DMA,
        ],
    )


def kernel(fixed_values, refinable_params, refinable_idx):
    idx2 = refinable_idx.astype(jnp.int32)
    par2 = refinable_params
    out = _tc_copy(fixed_values)
    out_ref = jax.new_ref(out.reshape(_N))
    _make_sc_scatter()(out_ref, idx2, par2)
    return out_ref[...].reshape(_ROWS, _COLS)


# speedup vs baseline: 12.3263x; 12.3263x over previous
import jax
import jax.numpy as jnp
from jax import lax
from jax.experimental import pallas as pl
from jax.experimental.pallas import tpu as pltpu
from jax.experimental.pallas import tpu_sc as plsc

_ROWS, _COLS = 16384, 1024
_N = _ROWS * _COLS
_R = _N // 4

_NC, _NS = 2, 16
_NW = _NC * _NS
_CS = 32768
_NCHUNK = _N // _CS
_CPW = _NCHUNK // _NW
_SU = 2048
_NB = 1024


def _sc_body(fix_ref, par_ref, idx_ref, bnd_ref, out_ref,
             buf, idx_w, par_w, bnd_v, sem):
    c = lax.axis_index("c")
    s = lax.axis_index("s")
    wid = s * _NC + c

    pltpu.sync_copy(bnd_ref, bnd_v)
    lane = lax.iota(jnp.int32, 16)

    def chunk_body(cl, carry):
        ch = wid * _CPW + cl
        lo = ch * _CS
        bv = bnd_v[pl.ds(ch, 16)]
        k0 = bv[0]
        k1 = bv[1]
        a0 = k0 & ~jnp.int32(7)
        nwin = lax.max(jnp.int32(0), (k1 - a0 + _SU - 1) // _SU)

        cp_in = pltpu.async_copy(fix_ref.at[pl.ds(lo, _CS)], buf, sem)
        cp_in.wait()

        def win_body(t, carry2):
            sft = pl.multiple_of(lax.min(a0 + t * _SU, jnp.int32(_R - _SU)), 8)
            pltpu.sync_copy(idx_ref.at[pl.ds(sft, _SU)], idx_w)
            pltpu.sync_copy(par_ref.at[pl.ds(sft, _SU)], par_w)

            def vec_body(v, carry3):
                q = sft + v * 16
                pos = q + lane
                ivec = idx_w[pl.ds(v * 16, 16)]
                pvec = par_w[pl.ds(v * 16, 16)]
                mask = (pos >= k0) & (pos < k1)
                plsc.store_scatter(buf, [ivec - lo], pvec, mask=mask)
                return carry3

            lax.fori_loop(0, _SU // 16, vec_body, 0, unroll=4)
            return carry2

        lax.fori_loop(0, nwin, win_body, 0)

        cp_out = pltpu.async_copy(buf, out_ref.at[pl.ds(lo, _CS)], sem)
        cp_out.wait()
        return carry

    lax.fori_loop(0, _CPW, chunk_body, 0)


def _make_sc_kernel():
    mesh = plsc.VectorSubcoreMesh(
        core_axis_name="c", subcore_axis_name="s",
        num_cores=_NC, num_subcores=_NS,
    )
    return pl.kernel(
        _sc_body,
        out_type=jax.ShapeDtypeStruct((_N,), jnp.float32),
        mesh=mesh,
        scratch_types=[
            pltpu.VMEM((_CS,), jnp.float32),
            pltpu.VMEM((_SU,), jnp.int32),
            pltpu.VMEM((_SU,), jnp.float32),
            pltpu.VMEM((_NB,), jnp.int32),
            pltpu.SemaphoreType.DMA,
        ],
        compiler_params=pltpu.CompilerParams(needs_layout_passes=False),
    )


def kernel(fixed_values, refinable_params, refinable_idx):
    idx32 = refinable_idx.astype(jnp.int32)
    cuts = jnp.arange(_NCHUNK + 1, dtype=jnp.int32) * _CS
    bnd = jnp.searchsorted(idx32, cuts, side="left").astype(jnp.int32)
    bnd = jnp.concatenate(
        [bnd, jnp.full((_NB - _NCHUNK - 1,), _R, dtype=jnp.int32)])
    out = _make_sc_kernel()(
        fixed_values.reshape(_N), refinable_params, idx32, bnd)
    return out.reshape(_ROWS, _COLS)
